# baseline (speedup 1.0000x reference)
import functools

import jax
import jax.numpy as jnp
from jax import lax
from jax.experimental import pallas as pl
from jax.experimental.pallas import tpu as pltpu
from jax.experimental.pallas import tpu_sc as plsc

N = 10000
E = 320000
D = 128
NLAYER = 6
NEG = 0.01
EPS = 1e-5

NC, NS = 2, 16
NW = NC * NS
NPAD = 10240
ZPT = NPAD // NS
K = 128
CHUNKS = 80
EPW = CHUNKS * K
EPAD = NW * EPW


SB = 14
MASK = (1 << SB) - 1


def _sc_body(mode, *refs):
    if mode == "agg":
        (h_hbm, pk_hbm, out_hbm, pk, isa, ida, isb, idb,
         rows_a, rows_b, acc_sh, sem_a, sem_b, ssem_a, ssem_b) = refs
    else:
        pk_hbm, out_hbm, pk, hist = refs
    c = lax.axis_index("c")
    s = lax.axis_index("s")
    wid = c * NS + s
    z16 = jnp.zeros((16,), jnp.float32)

    if mode == "deg":
        def _zh(i, _):
            hist[pl.ds(i * 16, 16)] = z16
            return 0
        lax.fori_loop(0, NPAD // 16, _zh, 0)
        pltpu.sync_copy(pk_hbm.at[wid], pk)
        o16 = jnp.ones((16,), jnp.float32)
        def _hstep(i, _):
            for j in range(K // 16):
                p = pk[pl.ds(i * K + j * 16, 16)]
                plsc.addupdate_scatter(hist, [lax.bitwise_and(p, MASK)], o16)
            return 0
        lax.fori_loop(0, CHUNKS, _hstep, 0)
        pltpu.sync_copy(hist, out_hbm.at[wid])
        return

    def _zrow(i, _):
        for j in range(D // 16):
            rows_a[i, pl.ds(j * 16, 16)] = z16
        return 0
    lax.fori_loop(0, K, _zrow, 0)
    z0 = s * ZPT
    for jj in range(ZPT // K):
        pltpu.sync_copy(rows_a, acc_sh.at[pl.ds(z0 + jj * K, K)])

    pltpu.sync_copy(pk_hbm.at[wid], pk)

    def _unpack(i, sbuf, dbuf):
        for j in range(K // 16):
            p = pk[i, pl.ds(j * 16, 16)]
            if sbuf is not None:
                sbuf[pl.ds(j * 16, 16)] = lax.shift_right_logical(p, SB)
            dbuf[pl.ds(j * 16, 16)] = lax.bitwise_and(p, MASK)

    plsc.subcore_barrier()

    _unpack(0, isa, ida)
    pltpu.async_copy(h_hbm.at[isa], rows_a, sem_a)
    _unpack(1, isb, idb)
    pltpu.async_copy(h_hbm.at[isb], rows_b, sem_b)

    def _step(j, _):
        i0 = 2 * j
        pltpu.make_async_copy(h_hbm.at[isa], rows_a, sem_a).wait()
        pltpu.async_copy(rows_a, acc_sh.at[ida], ssem_a, add=True)
        pltpu.make_async_copy(h_hbm.at[isb], rows_b, sem_b).wait()
        pltpu.async_copy(rows_b, acc_sh.at[idb], ssem_b, add=True)
        @pl.when(i0 + 2 < CHUNKS)
        def _():
            pltpu.make_async_copy(rows_a, acc_sh.at[ida], ssem_a).wait()
            _unpack(i0 + 2, isa, ida)
            pltpu.async_copy(h_hbm.at[isa], rows_a, sem_a)
        @pl.when(i0 + 3 < CHUNKS)
        def _():
            pltpu.make_async_copy(rows_b, acc_sh.at[idb], ssem_b).wait()
            _unpack(i0 + 3, isb, idb)
            pltpu.async_copy(h_hbm.at[isb], rows_b, sem_b)
        return 0
    lax.fori_loop(0, CHUNKS // 2, _step, 0)

    pltpu.make_async_copy(rows_a, acc_sh.at[ida], ssem_a).wait()
    pltpu.make_async_copy(rows_b, acc_sh.at[idb], ssem_b).wait()

    plsc.subcore_barrier()

    for jj in range(ZPT // K):
        sl = pl.ds(z0 + jj * K, K)
        pltpu.sync_copy(acc_sh.at[sl], out_hbm.at[c, sl])


@functools.cache
def _get_sc(mode):
    if mode == "agg":
        scratch = [
            pltpu.VMEM((CHUNKS, K), jnp.int32),
            pltpu.VMEM((K,), jnp.int32),
            pltpu.VMEM((K,), jnp.int32),
            pltpu.VMEM((K,), jnp.int32),
            pltpu.VMEM((K,), jnp.int32),
            pltpu.VMEM((K, D), jnp.float32),
            pltpu.VMEM((K, D), jnp.float32),
            pltpu.VMEM_SHARED((NPAD, D), jnp.float32),
            pltpu.SemaphoreType.DMA,
            pltpu.SemaphoreType.DMA,
            pltpu.SemaphoreType.DMA,
            pltpu.SemaphoreType.DMA,
        ]
        out_type = jax.ShapeDtypeStruct((NC, NPAD, D), jnp.float32)
    else:
        scratch = [
            pltpu.VMEM((CHUNKS * K,), jnp.int32),
            pltpu.VMEM((NPAD,), jnp.float32),
        ]
        out_type = jax.ShapeDtypeStruct((NW, NPAD), jnp.float32)
    mesh = plsc.VectorSubcoreMesh(core_axis_name="c", subcore_axis_name="s",
                                  num_cores=NC, num_subcores=NS)
    params = None
    if mode == "deg":
        params = pltpu.CompilerParams(needs_layout_passes=False)
    return pl.kernel(
        functools.partial(_sc_body, mode),
        out_type=out_type,
        mesh=mesh,
        scratch_types=scratch,
        compiler_params=params,
    )


def _spmm(h, packed):
    return _get_sc("agg")(h, packed)


def _deg(packed):
    return _get_sc("deg")(packed.reshape(NW, CHUNKS * K))


R = 1000
GRID = N // R


def _inv_body(dp_ref, o_ref):
    deg = lax.dot_general(dp_ref[...], jnp.ones((NW, D), jnp.float32),
                          (((0,), (0,)), ((), ())),
                          preferred_element_type=jnp.float32)
    o_ref[...] = 1.0 / jnp.maximum(deg, 1.0)


def _inv_deg(deg_p):
    rb = 1024
    return pl.pallas_call(
        _inv_body,
        grid=(NPAD // rb,),
        in_specs=[pl.BlockSpec((NW, rb), lambda i: (0, i))],
        out_specs=pl.BlockSpec((rb, D), lambda i: (i, 0)),
        out_shape=jax.ShapeDtypeStruct((NPAD, D), jnp.float32),
    )(deg_p)


def _dense_body(mode, *refs):
    if mode == "resid":
        h_ref, p_ref, inv_ref, wst, wnt, bias, g, bt, res_ref, o_ref = refs
    else:
        h_ref, p_ref, inv_ref, wst, wnt, bias, g, bt, o_ref = refs
    agg = (p_ref[0] + p_ref[1]) * inv_ref[...]
    hh = (jnp.dot(h_ref[...], wst[...], preferred_element_type=jnp.float32)
          + jnp.dot(agg, wnt[...], preferred_element_type=jnp.float32)
          + bias[...])
    hh = jnp.where(hh >= 0, hh, NEG * hh)
    mu = jnp.mean(hh, axis=-1, keepdims=True)
    xc = hh - mu
    var = jnp.mean(xc * xc, axis=-1, keepdims=True)
    y = g[...] * xc * lax.rsqrt(var + EPS) + bt[...]
    if mode == "resid":
        z = y + res_ref[...]
        mu2 = jnp.mean(z, axis=-1, keepdims=True)
        zc = z - mu2
        var2 = jnp.mean(zc * zc, axis=-1, keepdims=True)
        o_ref[...] = g[...] * zc * lax.rsqrt(var2 + EPS) + bt[...]
    elif mode == "pool":
        @pl.when(pl.program_id(0) == 0)
        def _():
            o_ref[...] = jnp.zeros_like(o_ref)
        o_ref[...] += jnp.sum(y, axis=0, keepdims=True) / N
    else:
        o_ref[...] = y


def _dense(h, agg_p, inv, wst, wnt, bias, g, bt, resid=None, pool=False):
    mode = "resid" if resid is not None else ("pool" if pool else "plain")
    row = lambda i: (i, 0)
    full = lambda i: (0, 0)
    in_specs = [
        pl.BlockSpec((R, D), row),
        pl.BlockSpec((NC, R, D), lambda i: (0, i, 0)),
        pl.BlockSpec((R, D), row),
        pl.BlockSpec((D, D), full),
        pl.BlockSpec((D, D), full),
        pl.BlockSpec((1, D), full),
        pl.BlockSpec((1, D), full),
        pl.BlockSpec((1, D), full),
    ]
    args = [h, agg_p, inv, wst, wnt, bias, g, bt]
    if resid is not None:
        in_specs.append(pl.BlockSpec((R, D), row))
        args.append(resid)
    if pool:
        out_spec = pl.BlockSpec((1, D), full)
        out_shape = jax.ShapeDtypeStruct((1, D), jnp.float32)
    else:
        out_spec = pl.BlockSpec((R, D), row)
        out_shape = jax.ShapeDtypeStruct((N, D), jnp.float32)
    return pl.pallas_call(
        functools.partial(_dense_body, mode),
        grid=(GRID,),
        in_specs=in_specs,
        out_specs=out_spec,
        out_shape=out_shape,
    )(*args)


def kernel(feats, edge_index, W_self, W_neigh, b, ln_gamma, ln_beta, training=False):
    src = edge_index[0].astype(jnp.int32)
    dst = edge_index[1].astype(jnp.int32)
    pad = EPAD - E
    ps = jnp.arange(pad, dtype=jnp.int32) % N
    pd = N + jnp.arange(pad, dtype=jnp.int32) % (NPAD - N)
    srcp = jnp.concatenate([src, ps])
    dstp = jnp.concatenate([dst, pd])
    packed = ((srcp << SB) | dstp).reshape(NW, CHUNKS, K)

    wst = jnp.swapaxes(W_self, 1, 2)
    wnt = jnp.swapaxes(W_neigh, 1, 2)
    bias = b.reshape(NLAYER, 1, D)
    g = ln_gamma.reshape(1, D)
    bt = ln_beta.reshape(1, D)

    inv = _inv_deg(_deg(packed))
    h = feats
    for li in range(NLAYER):
        agg_p = _spmm(h, packed)
        h = _dense(h, agg_p, inv, wst[li], wnt[li], bias[li], g, bt,
                   resid=feats if li == 2 else None,
                   pool=(li == NLAYER - 1))
    return h


# speedup vs baseline: 1.3057x; 1.3057x over previous
import functools

import jax
import jax.numpy as jnp
from jax import lax
from jax.experimental import pallas as pl
from jax.experimental.pallas import tpu as pltpu
from jax.experimental.pallas import tpu_sc as plsc

N = 10000
E = 320000
D = 128
NLAYER = 6
NEG = 0.01
EPS = 1e-5

NC, NS = 2, 16
NW = NC * NS
NPAD = 10240
ZPT = NPAD // NS
K = 128
CHUNKS = 80
EPW = CHUNKS * K
EPAD = NW * EPW


SB = 14
MASK = (1 << SB) - 1


def _sc_body(mode, *refs):
    if mode == "agg":
        (h_hbm, pk_hbm, out_hbm, pk, isa, ida, isb, idb,
         rows_a, rows_b, acc_sh, sem_a, sem_b) = refs
    else:
        pk_hbm, out_hbm, pk, hist = refs
    c = lax.axis_index("c")
    s = lax.axis_index("s")
    wid = c * NS + s
    z16 = jnp.zeros((16,), jnp.float32)

    if mode == "deg":
        def _zh(i, _):
            hist[pl.ds(i * 16, 16)] = z16
            return 0
        lax.fori_loop(0, NPAD // 16, _zh, 0)
        pltpu.sync_copy(pk_hbm.at[wid], pk)
        o16 = jnp.ones((16,), jnp.float32)
        def _hstep(i, _):
            for j in range(K // 16):
                p = pk[pl.ds(i * K + j * 16, 16)]
                plsc.addupdate_scatter(hist, [lax.bitwise_and(p, MASK)], o16)
            return 0
        lax.fori_loop(0, CHUNKS, _hstep, 0)
        pltpu.sync_copy(hist, out_hbm.at[wid])
        return

    def _zrow(i, _):
        for j in range(D // 16):
            rows_a[i, pl.ds(j * 16, 16)] = z16
        return 0
    lax.fori_loop(0, K, _zrow, 0)
    z0 = s * ZPT
    for jj in range(ZPT // K):
        pltpu.sync_copy(rows_a, acc_sh.at[pl.ds(z0 + jj * K, K)])

    pltpu.sync_copy(pk_hbm.at[wid], pk)

    def _unpack(i, sbuf, dbuf):
        for j in range(K // 16):
            p = pk[i, pl.ds(j * 16, 16)]
            if sbuf is not None:
                sbuf[pl.ds(j * 16, 16)] = lax.shift_right_logical(p, SB)
            dbuf[pl.ds(j * 16, 16)] = lax.bitwise_and(p, MASK)

    plsc.subcore_barrier()

    _unpack(0, isa, ida)
    pltpu.async_copy(h_hbm.at[isa], rows_a, sem_a)
    _unpack(1, isb, idb)
    pltpu.async_copy(h_hbm.at[isb], rows_b, sem_b)

    def _half(j, rows, sem, isx, idx, off):
        i2 = 2 * j + off + 2
        pltpu.make_async_copy(h_hbm.at[isx], rows, sem).wait()
        pltpu.sync_copy(rows, acc_sh.at[idx], add=True)
        @pl.when(i2 < CHUNKS)
        def _():
            _unpack(i2, isx, idx)
            pltpu.async_copy(h_hbm.at[isx], rows, sem)

    def _step(j, _):
        _half(j, rows_a, sem_a, isa, ida, 0)
        _half(j, rows_b, sem_b, isb, idb, 1)
        return 0
    lax.fori_loop(0, CHUNKS // 2, _step, 0)

    plsc.subcore_barrier()

    for jj in range(ZPT // K):
        sl = pl.ds(z0 + jj * K, K)
        pltpu.sync_copy(acc_sh.at[sl], out_hbm.at[c, sl])


@functools.cache
def _get_sc(mode):
    if mode == "agg":
        scratch = [
            pltpu.VMEM((CHUNKS, K), jnp.int32),
            pltpu.VMEM((K,), jnp.int32),
            pltpu.VMEM((K,), jnp.int32),
            pltpu.VMEM((K,), jnp.int32),
            pltpu.VMEM((K,), jnp.int32),
            pltpu.VMEM((K, D), jnp.float32),
            pltpu.VMEM((K, D), jnp.float32),
            pltpu.VMEM_SHARED((NPAD, D), jnp.float32),
            pltpu.SemaphoreType.DMA,
            pltpu.SemaphoreType.DMA,
        ]
        out_type = jax.ShapeDtypeStruct((NC, NPAD, D), jnp.float32)
    else:
        scratch = [
            pltpu.VMEM((CHUNKS * K,), jnp.int32),
            pltpu.VMEM((NPAD,), jnp.float32),
        ]
        out_type = jax.ShapeDtypeStruct((NW, NPAD), jnp.float32)
    mesh = plsc.VectorSubcoreMesh(core_axis_name="c", subcore_axis_name="s",
                                  num_cores=NC, num_subcores=NS)
    params = None
    if mode == "deg":
        params = pltpu.CompilerParams(needs_layout_passes=False)
    return pl.kernel(
        functools.partial(_sc_body, mode),
        out_type=out_type,
        mesh=mesh,
        scratch_types=scratch,
        compiler_params=params,
    )


def _spmm(h, packed):
    return _get_sc("agg")(h, packed)


def _deg(packed):
    return _get_sc("deg")(packed.reshape(NW, CHUNKS * K))


R = 1000
GRID = N // R


def _inv_body(dp_ref, o_ref):
    deg = lax.dot_general(dp_ref[...], jnp.ones((NW, D), jnp.float32),
                          (((0,), (0,)), ((), ())),
                          preferred_element_type=jnp.float32)
    o_ref[...] = 1.0 / jnp.maximum(deg, 1.0)


def _inv_deg(deg_p):
    rb = 1024
    return pl.pallas_call(
        _inv_body,
        grid=(NPAD // rb,),
        in_specs=[pl.BlockSpec((NW, rb), lambda i: (0, i))],
        out_specs=pl.BlockSpec((rb, D), lambda i: (i, 0)),
        out_shape=jax.ShapeDtypeStruct((NPAD, D), jnp.float32),
    )(deg_p)


def _dense_body(mode, *refs):
    if mode == "resid":
        h_ref, p_ref, inv_ref, wst, wnt, bias, g, bt, res_ref, o_ref = refs
    else:
        h_ref, p_ref, inv_ref, wst, wnt, bias, g, bt, o_ref = refs
    agg = (p_ref[0] + p_ref[1]) * inv_ref[...]
    hh = (jnp.dot(h_ref[...], wst[...], preferred_element_type=jnp.float32)
          + jnp.dot(agg, wnt[...], preferred_element_type=jnp.float32)
          + bias[...])
    hh = jnp.where(hh >= 0, hh, NEG * hh)
    mu = jnp.mean(hh, axis=-1, keepdims=True)
    xc = hh - mu
    var = jnp.mean(xc * xc, axis=-1, keepdims=True)
    y = g[...] * xc * lax.rsqrt(var + EPS) + bt[...]
    if mode == "resid":
        z = y + res_ref[...]
        mu2 = jnp.mean(z, axis=-1, keepdims=True)
        zc = z - mu2
        var2 = jnp.mean(zc * zc, axis=-1, keepdims=True)
        o_ref[...] = g[...] * zc * lax.rsqrt(var2 + EPS) + bt[...]
    elif mode == "pool":
        @pl.when(pl.program_id(0) == 0)
        def _():
            o_ref[...] = jnp.zeros_like(o_ref)
        o_ref[...] += jnp.sum(y, axis=0, keepdims=True) / N
    else:
        o_ref[...] = y


def _dense(h, agg_p, inv, wst, wnt, bias, g, bt, resid=None, pool=False):
    mode = "resid" if resid is not None else ("pool" if pool else "plain")
    row = lambda i: (i, 0)
    full = lambda i: (0, 0)
    in_specs = [
        pl.BlockSpec((R, D), row),
        pl.BlockSpec((NC, R, D), lambda i: (0, i, 0)),
        pl.BlockSpec((R, D), row),
        pl.BlockSpec((D, D), full),
        pl.BlockSpec((D, D), full),
        pl.BlockSpec((1, D), full),
        pl.BlockSpec((1, D), full),
        pl.BlockSpec((1, D), full),
    ]
    args = [h, agg_p, inv, wst, wnt, bias, g, bt]
    if resid is not None:
        in_specs.append(pl.BlockSpec((R, D), row))
        args.append(resid)
    if pool:
        out_spec = pl.BlockSpec((1, D), full)
        out_shape = jax.ShapeDtypeStruct((1, D), jnp.float32)
    else:
        out_spec = pl.BlockSpec((R, D), row)
        out_shape = jax.ShapeDtypeStruct((N, D), jnp.float32)
    return pl.pallas_call(
        functools.partial(_dense_body, mode),
        grid=(GRID,),
        in_specs=in_specs,
        out_specs=out_spec,
        out_shape=out_shape,
    )(*args)


def kernel(feats, edge_index, W_self, W_neigh, b, ln_gamma, ln_beta, training=False):
    src = edge_index[0].astype(jnp.int32)
    dst = edge_index[1].astype(jnp.int32)
    pad = EPAD - E
    ps = jnp.arange(pad, dtype=jnp.int32) % N
    pd = N + jnp.arange(pad, dtype=jnp.int32) % (NPAD - N)
    srcp = jnp.concatenate([src, ps])
    dstp = jnp.concatenate([dst, pd])
    packed = ((srcp << SB) | dstp).reshape(NW, CHUNKS, K)

    wst = jnp.swapaxes(W_self, 1, 2)
    wnt = jnp.swapaxes(W_neigh, 1, 2)
    bias = b.reshape(NLAYER, 1, D)
    g = ln_gamma.reshape(1, D)
    bt = ln_beta.reshape(1, D)

    inv = _inv_deg(_deg(packed))
    h = feats
    for li in range(NLAYER):
        agg_p = _spmm(h, packed)
        h = _dense(h, agg_p, inv, wst[li], wnt[li], bias[li], g, bt,
                   resid=feats if li == 2 else None,
                   pool=(li == NLAYER - 1))
    return h


# speedup vs baseline: 1.3259x; 1.0154x over previous
import functools

import jax
import jax.numpy as jnp
from jax import lax
from jax.experimental import pallas as pl
from jax.experimental.pallas import tpu as pltpu
from jax.experimental.pallas import tpu_sc as plsc

N = 10000
E = 320000
D = 128
NLAYER = 6
NEG = 0.01
EPS = 1e-5

NC, NS = 2, 16
NW = NC * NS
NPAD = 10240
ZPT = NPAD // NS
K = 128
CHUNKS = 80
EPW = CHUNKS * K
EPAD = NW * EPW


SB = 14
MASK = (1 << SB) - 1


def _sc_body(mode, *refs):
    if mode == "agg":
        (h_hbm, pk_hbm, out_hbm, pk, isa, ida, isb, idb,
         rows_a, rows_b, acc_sh, sem_a, sem_b) = refs
    else:
        pk_hbm, out_hbm, pk, hist = refs
    c = lax.axis_index("c")
    s = lax.axis_index("s")
    wid = c * NS + s
    z16 = jnp.zeros((16,), jnp.float32)

    if mode == "deg":
        def _zh(i, _):
            hist[pl.ds(i * 16, 16)] = z16
            return 0
        lax.fori_loop(0, NPAD // 16, _zh, 0)
        pltpu.sync_copy(pk_hbm.at[wid], pk)
        o16 = jnp.ones((16,), jnp.float32)
        def _hstep(i, _):
            for j in range(K // 16):
                p = pk[pl.ds(i * K + j * 16, 16)]
                plsc.addupdate_scatter(hist, [lax.bitwise_and(p, MASK)], o16)
            return 0
        lax.fori_loop(0, CHUNKS, _hstep, 0)
        pltpu.sync_copy(hist, out_hbm.at[wid])
        return

    pltpu.sync_copy(pk_hbm.at[wid], pk)

    def _unpack(i, sbuf, dbuf):
        for j in range(K // 16):
            p = pk[i, pl.ds(j * 16, 16)]
            if sbuf is not None:
                sbuf[pl.ds(j * 16, 16)] = lax.shift_right_logical(p, SB)
            dbuf[pl.ds(j * 16, 16)] = lax.bitwise_and(p, MASK)

    _unpack(0, isa, ida)
    pltpu.async_copy(h_hbm.at[isa], rows_a, sem_a)
    def _zrow(i, _):
        for j in range(D // 16):
            rows_b[i, pl.ds(j * 16, 16)] = z16
        return 0
    lax.fori_loop(0, K, _zrow, 0)
    z0 = s * ZPT
    for jj in range(ZPT // K):
        pltpu.sync_copy(rows_b, acc_sh.at[pl.ds(z0 + jj * K, K)])
    _unpack(1, isb, idb)
    pltpu.async_copy(h_hbm.at[isb], rows_b, sem_b)

    plsc.subcore_barrier()

    def _half(j, rows, sem, isx, idx, off):
        i2 = 2 * j + off + 2
        pltpu.make_async_copy(h_hbm.at[isx], rows, sem).wait()
        pltpu.sync_copy(rows, acc_sh.at[idx], add=True)
        @pl.when(i2 < CHUNKS)
        def _():
            _unpack(i2, isx, idx)
            pltpu.async_copy(h_hbm.at[isx], rows, sem)

    def _step(j, _):
        _half(j, rows_a, sem_a, isa, ida, 0)
        _half(j, rows_b, sem_b, isb, idb, 1)
        return 0
    lax.fori_loop(0, CHUNKS // 2, _step, 0)

    plsc.subcore_barrier()

    for jj in range(ZPT // K):
        sl = pl.ds(z0 + jj * K, K)
        pltpu.sync_copy(acc_sh.at[sl], out_hbm.at[c, sl])


@functools.cache
def _get_sc(mode):
    if mode == "agg":
        scratch = [
            pltpu.VMEM((CHUNKS, K), jnp.int32),
            pltpu.VMEM((K,), jnp.int32),
            pltpu.VMEM((K,), jnp.int32),
            pltpu.VMEM((K,), jnp.int32),
            pltpu.VMEM((K,), jnp.int32),
            pltpu.VMEM((K, D), jnp.float32),
            pltpu.VMEM((K, D), jnp.float32),
            pltpu.VMEM_SHARED((NPAD, D), jnp.float32),
            pltpu.SemaphoreType.DMA,
            pltpu.SemaphoreType.DMA,
        ]
        out_type = jax.ShapeDtypeStruct((NC, NPAD, D), jnp.float32)
    else:
        scratch = [
            pltpu.VMEM((CHUNKS * K,), jnp.int32),
            pltpu.VMEM((NPAD,), jnp.float32),
        ]
        out_type = jax.ShapeDtypeStruct((NW, NPAD), jnp.float32)
    mesh = plsc.VectorSubcoreMesh(core_axis_name="c", subcore_axis_name="s",
                                  num_cores=NC, num_subcores=NS)
    params = None
    if mode == "deg":
        params = pltpu.CompilerParams(needs_layout_passes=False)
    return pl.kernel(
        functools.partial(_sc_body, mode),
        out_type=out_type,
        mesh=mesh,
        scratch_types=scratch,
        compiler_params=params,
    )


def _spmm(h, packed):
    return _get_sc("agg")(h, packed)


def _deg(packed):
    return _get_sc("deg")(packed.reshape(NW, CHUNKS * K))


R = 1000
GRID = N // R


def _inv_body(dp_ref, o_ref):
    deg = lax.dot_general(dp_ref[...], jnp.ones((NW, D), jnp.float32),
                          (((0,), (0,)), ((), ())),
                          preferred_element_type=jnp.float32)
    o_ref[...] = 1.0 / jnp.maximum(deg, 1.0)


def _inv_deg(deg_p):
    rb = 1024
    return pl.pallas_call(
        _inv_body,
        grid=(NPAD // rb,),
        in_specs=[pl.BlockSpec((NW, rb), lambda i: (0, i))],
        out_specs=pl.BlockSpec((rb, D), lambda i: (i, 0)),
        out_shape=jax.ShapeDtypeStruct((NPAD, D), jnp.float32),
    )(deg_p)


def _dense_body(mode, *refs):
    if mode == "resid":
        h_ref, p_ref, inv_ref, wst, wnt, bias, g, bt, res_ref, o_ref = refs
    else:
        h_ref, p_ref, inv_ref, wst, wnt, bias, g, bt, o_ref = refs
    agg = (p_ref[0] + p_ref[1]) * inv_ref[...]
    hh = (jnp.dot(h_ref[...], wst[...], preferred_element_type=jnp.float32)
          + jnp.dot(agg, wnt[...], preferred_element_type=jnp.float32)
          + bias[...])
    hh = jnp.where(hh >= 0, hh, NEG * hh)
    mu = jnp.mean(hh, axis=-1, keepdims=True)
    xc = hh - mu
    var = jnp.mean(xc * xc, axis=-1, keepdims=True)
    y = g[...] * xc * lax.rsqrt(var + EPS) + bt[...]
    if mode == "resid":
        z = y + res_ref[...]
        mu2 = jnp.mean(z, axis=-1, keepdims=True)
        zc = z - mu2
        var2 = jnp.mean(zc * zc, axis=-1, keepdims=True)
        o_ref[...] = g[...] * zc * lax.rsqrt(var2 + EPS) + bt[...]
    elif mode == "pool":
        @pl.when(pl.program_id(0) == 0)
        def _():
            o_ref[...] = jnp.zeros_like(o_ref)
        o_ref[...] += jnp.sum(y, axis=0, keepdims=True) / N
    else:
        o_ref[...] = y


def _dense(h, agg_p, inv, wst, wnt, bias, g, bt, resid=None, pool=False):
    mode = "resid" if resid is not None else ("pool" if pool else "plain")
    row = lambda i: (i, 0)
    full = lambda i: (0, 0)
    in_specs = [
        pl.BlockSpec((R, D), row),
        pl.BlockSpec((NC, R, D), lambda i: (0, i, 0)),
        pl.BlockSpec((R, D), row),
        pl.BlockSpec((D, D), full),
        pl.BlockSpec((D, D), full),
        pl.BlockSpec((1, D), full),
        pl.BlockSpec((1, D), full),
        pl.BlockSpec((1, D), full),
    ]
    args = [h, agg_p, inv, wst, wnt, bias, g, bt]
    if resid is not None:
        in_specs.append(pl.BlockSpec((R, D), row))
        args.append(resid)
    if pool:
        out_spec = pl.BlockSpec((1, D), full)
        out_shape = jax.ShapeDtypeStruct((1, D), jnp.float32)
    else:
        out_spec = pl.BlockSpec((R, D), row)
        out_shape = jax.ShapeDtypeStruct((N, D), jnp.float32)
    return pl.pallas_call(
        functools.partial(_dense_body, mode),
        grid=(GRID,),
        in_specs=in_specs,
        out_specs=out_spec,
        out_shape=out_shape,
    )(*args)


def kernel(feats, edge_index, W_self, W_neigh, b, ln_gamma, ln_beta, training=False):
    src = edge_index[0].astype(jnp.int32)
    dst = edge_index[1].astype(jnp.int32)
    pad = EPAD - E
    ps = jnp.arange(pad, dtype=jnp.int32) % N
    pd = N + jnp.arange(pad, dtype=jnp.int32) % (NPAD - N)
    srcp = jnp.concatenate([src, ps])
    dstp = jnp.concatenate([dst, pd])
    packed = ((srcp << SB) | dstp).reshape(NW, CHUNKS, K)

    wst = jnp.swapaxes(W_self, 1, 2)
    wnt = jnp.swapaxes(W_neigh, 1, 2)
    bias = b.reshape(NLAYER, 1, D)
    g = ln_gamma.reshape(1, D)
    bt = ln_beta.reshape(1, D)

    inv = _inv_deg(_deg(packed))
    h = feats
    for li in range(NLAYER):
        agg_p = _spmm(h, packed)
        h = _dense(h, agg_p, inv, wst[li], wnt[li], bias[li], g, bt,
                   resid=feats if li == 2 else None,
                   pool=(li == NLAYER - 1))
    return h
